# Initial kernel scaffold; baseline (speedup 1.0000x reference)
import functools

import jax
import jax.numpy as jnp
from jax import lax
from jax.experimental import pallas as pl
from jax.experimental.pallas import tpu as pltpu
from jax.experimental.pallas import tpu_sc as plsc

E = 8
K = 2
H = 1024
I = 2048
T = 4096

BM = 256
NB = (T * K) // BM + E
P = NB * BM

NC = 2
NS = 16
NW = NC * NS

G_CH = 32
G_NCH = P // (NW * G_CH)
G_ROWS_W = P // NW


@functools.partial(
    pl.kernel,
    out_type=jax.ShapeDtypeStruct((P, H), jnp.float32),
    mesh=plsc.VectorSubcoreMesh(core_axis_name="c", subcore_axis_name="s"),
    scratch_types=[
        pltpu.VMEM((G_NCH, G_CH), jnp.int32),
        pltpu.VMEM((G_CH, H), jnp.float32),
        pltpu.VMEM((G_CH, H), jnp.float32),
        pltpu.SemaphoreType.DMA,
        pltpu.SemaphoreType.DMA,
    ],
)
def _sc_gather(x_hbm, ids_hbm, out_hbm, idx_v, buf0, buf1, s0, s1):
    wid = lax.axis_index("s") * NC + lax.axis_index("c")
    rbase = wid * G_ROWS_W
    pltpu.sync_copy(ids_hbm.at[pl.ds(wid * G_NCH, G_NCH)], idx_v)
    bufs = (buf0, buf1)
    sems = (s0, s1)
    cps = [None, None]
    cps[0] = pltpu.async_copy(x_hbm.at[idx_v.at[0]], buf0, s0)
    for c in range(G_NCH):
        if c + 1 < G_NCH:
            cps[(c + 1) % 2] = pltpu.async_copy(
                x_hbm.at[idx_v.at[c + 1]], bufs[(c + 1) % 2], sems[(c + 1) % 2]
            )
        cps[c % 2].wait()
        pltpu.sync_copy(bufs[c % 2], out_hbm.at[pl.ds(rbase + c * G_CH, G_CH)])


C_CH = 16
C_NCH = T // (NW * C_CH)
C_TOK_W = T // NW


@functools.partial(
    pl.kernel,
    out_type=jax.ShapeDtypeStruct((T, H), jnp.float32),
    mesh=plsc.VectorSubcoreMesh(core_axis_name="c", subcore_axis_name="s"),
    scratch_types=[
        pltpu.VMEM((C_NCH, C_CH), jnp.int32),
        pltpu.VMEM((C_NCH, C_CH), jnp.int32),
        pltpu.VMEM((C_CH, H), jnp.float32),
        pltpu.VMEM((C_CH, H), jnp.float32),
        pltpu.SemaphoreType.DMA,
        pltpu.SemaphoreType.DMA,
    ],
)
def _sc_combine(yw_hbm, i0_hbm, i1_hbm, out_hbm, i0v, i1v, a_buf, b_buf, s0, s1):
    wid = lax.axis_index("s") * NC + lax.axis_index("c")
    tbase = wid * C_TOK_W
    pltpu.sync_copy(i0_hbm.at[pl.ds(wid * C_NCH, C_NCH)], i0v)
    pltpu.sync_copy(i1_hbm.at[pl.ds(wid * C_NCH, C_NCH)], i1v)
    for c in range(C_NCH):
        ca = pltpu.async_copy(yw_hbm.at[i0v.at[c]], a_buf, s0)
        cb = pltpu.async_copy(yw_hbm.at[i1v.at[c]], b_buf, s1)
        ca.wait()
        cb.wait()
        for r in range(C_CH):
            def _add(j, _, r=r):
                sl = pl.ds(j * 16, 16)
                a_buf[r, sl] = a_buf[r, sl] + b_buf[r, sl]
                return 0
            lax.fori_loop(0, H // 16, _add, 0)
        pltpu.sync_copy(a_buf, out_hbm.at[pl.ds(tbase + c * C_CH, C_CH)])


def _mlp_body(s_ref, x_ref, w1_ref, w3_ref, w2_ref, rw_ref, out_ref):
    x = x_ref[...].astype(jnp.bfloat16)
    w1 = w1_ref[0].astype(jnp.bfloat16)
    w3 = w3_ref[0].astype(jnp.bfloat16)
    w2 = w2_ref[0].astype(jnp.bfloat16)
    dn = (((1,), (1,)), ((), ()))
    g = lax.dot_general(x, w1, dn, preferred_element_type=jnp.float32)
    u = lax.dot_general(x, w3, dn, preferred_element_type=jnp.float32)
    act = (g * jax.nn.sigmoid(g) * u).astype(jnp.bfloat16)
    y = lax.dot_general(act, w2, dn, preferred_element_type=jnp.float32)
    out_ref[...] = y * rw_ref[0, 0, :][:, None]


_tc_mlp = pl.pallas_call(
    _mlp_body,
    grid_spec=pltpu.PrefetchScalarGridSpec(
        num_scalar_prefetch=1,
        grid=(NB,),
        in_specs=[
            pl.BlockSpec((BM, H), lambda b, s: (b, 0)),
            pl.BlockSpec((1, I, H), lambda b, s: (s[b], 0, 0)),
            pl.BlockSpec((1, I, H), lambda b, s: (s[b], 0, 0)),
            pl.BlockSpec((1, H, I), lambda b, s: (s[b], 0, 0)),
            pl.BlockSpec((1, 1, BM), lambda b, s: (b, 0, 0)),
        ],
        out_specs=pl.BlockSpec((BM, H), lambda b, s: (b, 0)),
    ),
    out_shape=jax.ShapeDtypeStruct((P, H), jnp.float32),
    compiler_params=pltpu.CompilerParams(
        dimension_semantics=("arbitrary",),
        vmem_limit_bytes=100 * 1024 * 1024,
    ),
)


def _plan(router_logits):
    probs = jax.nn.softmax(router_logits, axis=-1)
    rw, sel = lax.top_k(probs, K)
    flat_e = sel.reshape(-1).astype(jnp.int32)
    flat_w = rw.reshape(-1).astype(jnp.float32)
    flat_t = (jnp.arange(T * K, dtype=jnp.int32) // K)
    counts = jnp.bincount(flat_e, length=E)
    pc = ((counts + BM - 1) // BM) * BM
    pad_end = jnp.cumsum(pc)
    pad_off = pad_end - pc
    off = jnp.cumsum(counts) - counts
    order = jnp.argsort(flat_e, stable=True)
    g_sorted = flat_e[order]
    s = jnp.arange(T * K, dtype=jnp.int32)
    dst_sorted = (pad_off[g_sorted] + s - off[g_sorted]).astype(jnp.int32)
    row_token = jnp.zeros((P,), jnp.int32).at[dst_sorted].set(flat_t[order])
    row_w = jnp.zeros((P,), jnp.float32).at[dst_sorted].set(flat_w[order])
    dst_pair = jnp.zeros((T * K,), jnp.int32).at[order].set(dst_sorted)
    inv = dst_pair.reshape(T, K)
    block_start = jnp.arange(NB, dtype=jnp.int32) * BM
    block_expert = jnp.minimum(
        jnp.searchsorted(pad_end, block_start, side="right"), E - 1
    ).astype(jnp.int32)
    return row_token, row_w, inv, block_expert


def kernel(hidden_states, router_logits, w1, w3, w2):
    x = hidden_states.reshape(T, H)
    row_token, row_w, inv, block_expert = _plan(router_logits)

    x_pad = _sc_gather(x, row_token.reshape(P // G_CH, G_CH))
    yw = _tc_mlp(block_expert, x_pad, w1, w3, w2, row_w.reshape(NB, 1, BM))
    out = _sc_combine(
        yw,
        inv[:, 0].reshape(T // C_CH, C_CH),
        inv[:, 1].reshape(T // C_CH, C_CH),
    )
    return out


# speedup vs baseline: 1.2426x; 1.2426x over previous
import functools

import jax
import jax.numpy as jnp
from jax import lax
from jax.experimental import pallas as pl
from jax.experimental.pallas import tpu as pltpu
from jax.experimental.pallas import tpu_sc as plsc

E = 8
K = 2
H = 1024
I = 2048
T = 4096

BM = 256
NB = (T * K) // BM + E
P = NB * BM

NC = 2
NS = 16
NW = NC * NS

G_CH = 32
G_NCH = P // (NW * G_CH)
G_ROWS_W = P // NW


def _sc_gather_body(x_hbm, ids_hbm, out_hbm, idx_v, buf0, buf1, s0, s1):
    wid = lax.axis_index("s") * NC + lax.axis_index("c")
    rbase = wid * G_ROWS_W
    pltpu.sync_copy(ids_hbm.at[wid], idx_v)
    bufs = (buf0, buf1)
    sems = (s0, s1)
    cps = [None, None]
    cps[0] = pltpu.async_copy(x_hbm.at[idx_v.at[0]], buf0, s0)
    for c in range(G_NCH):
        if c + 1 < G_NCH:
            cps[(c + 1) % 2] = pltpu.async_copy(
                x_hbm.at[idx_v.at[c + 1]], bufs[(c + 1) % 2], sems[(c + 1) % 2]
            )
        cps[c % 2].wait()
        pltpu.sync_copy(bufs[c % 2], out_hbm.at[pl.ds(rbase + c * G_CH, G_CH)])


C_CH = 16
C_NCH = T // (NW * C_CH)
C_TOK_W = T // NW


def _sc_combine_body(yw_hbm, i0_hbm, i1_hbm, out_hbm, i0v, i1v, a_buf, b_buf, s0, s1):
    wid = lax.axis_index("s") * NC + lax.axis_index("c")
    tbase = wid * C_TOK_W
    pltpu.sync_copy(i0_hbm.at[wid], i0v)
    pltpu.sync_copy(i1_hbm.at[wid], i1v)
    for c in range(C_NCH):
        ca = pltpu.async_copy(yw_hbm.at[i0v.at[c]], a_buf, s0)
        cb = pltpu.async_copy(yw_hbm.at[i1v.at[c]], b_buf, s1)
        ca.wait()
        cb.wait()
        for r in range(C_CH):
            def _add(j, _, r=r):
                sl = pl.ds(j * 16, 16)
                a_buf[r, sl] = a_buf[r, sl] + b_buf[r, sl]
                return 0
            lax.fori_loop(0, H // 16, _add, 0)
        pltpu.sync_copy(a_buf, out_hbm.at[pl.ds(tbase + c * C_CH, C_CH)])


@functools.lru_cache(maxsize=None)
def _sc_kernels():
    mesh = plsc.VectorSubcoreMesh(
        core_axis_name="c", subcore_axis_name="s", num_cores=NC, num_subcores=NS
    )
    gather = pl.kernel(
        _sc_gather_body,
        out_type=jax.ShapeDtypeStruct((P, H), jnp.float32),
        mesh=mesh,
        scratch_types=[
            pltpu.VMEM((G_NCH, G_CH), jnp.int32),
            pltpu.VMEM((G_CH, H), jnp.float32),
            pltpu.VMEM((G_CH, H), jnp.float32),
            pltpu.SemaphoreType.DMA,
            pltpu.SemaphoreType.DMA,
        ],
    )
    combine = pl.kernel(
        _sc_combine_body,
        out_type=jax.ShapeDtypeStruct((T, H), jnp.float32),
        mesh=mesh,
        scratch_types=[
            pltpu.VMEM((C_NCH, C_CH), jnp.int32),
            pltpu.VMEM((C_NCH, C_CH), jnp.int32),
            pltpu.VMEM((C_CH, H), jnp.float32),
            pltpu.VMEM((C_CH, H), jnp.float32),
            pltpu.SemaphoreType.DMA,
            pltpu.SemaphoreType.DMA,
        ],
    )
    return gather, combine


def _mlp_body(s_ref, x_ref, w1_ref, w3_ref, w2_ref, rw_ref, out_ref):
    x = x_ref[...].astype(jnp.bfloat16)
    w1 = w1_ref[0].astype(jnp.bfloat16)
    w3 = w3_ref[0].astype(jnp.bfloat16)
    w2 = w2_ref[0].astype(jnp.bfloat16)
    dn = (((1,), (1,)), ((), ()))
    g = lax.dot_general(x, w1, dn, preferred_element_type=jnp.float32)
    u = lax.dot_general(x, w3, dn, preferred_element_type=jnp.float32)
    act = (g * jax.nn.sigmoid(g) * u).astype(jnp.bfloat16)
    y = lax.dot_general(act, w2, dn, preferred_element_type=jnp.float32)
    out_ref[...] = y * rw_ref[0, 0, :][:, None]


_tc_mlp = pl.pallas_call(
    _mlp_body,
    grid_spec=pltpu.PrefetchScalarGridSpec(
        num_scalar_prefetch=1,
        grid=(NB,),
        in_specs=[
            pl.BlockSpec((BM, H), lambda b, s: (b, 0)),
            pl.BlockSpec((1, I, H), lambda b, s: (s[b], 0, 0)),
            pl.BlockSpec((1, I, H), lambda b, s: (s[b], 0, 0)),
            pl.BlockSpec((1, H, I), lambda b, s: (s[b], 0, 0)),
            pl.BlockSpec((1, 1, BM), lambda b, s: (b, 0, 0)),
        ],
        out_specs=pl.BlockSpec((BM, H), lambda b, s: (b, 0)),
    ),
    out_shape=jax.ShapeDtypeStruct((P, H), jnp.float32),
    compiler_params=pltpu.CompilerParams(
        dimension_semantics=("arbitrary",),
        vmem_limit_bytes=100 * 1024 * 1024,
    ),
)


def _plan(router_logits):
    probs = jax.nn.softmax(router_logits, axis=-1)
    rw, sel = lax.top_k(probs, K)
    flat_e = sel.reshape(-1).astype(jnp.int32)
    flat_w = rw.reshape(-1).astype(jnp.float32)
    flat_t = (jnp.arange(T * K, dtype=jnp.int32) // K)
    counts = jnp.bincount(flat_e, length=E)
    pc = ((counts + BM - 1) // BM) * BM
    pad_end = jnp.cumsum(pc)
    pad_off = pad_end - pc
    off = jnp.cumsum(counts) - counts
    order = jnp.argsort(flat_e, stable=True)
    g_sorted = flat_e[order]
    s = jnp.arange(T * K, dtype=jnp.int32)
    dst_sorted = (pad_off[g_sorted] + s - off[g_sorted]).astype(jnp.int32)
    row_token = jnp.zeros((P,), jnp.int32).at[dst_sorted].set(flat_t[order])
    row_w = jnp.zeros((P,), jnp.float32).at[dst_sorted].set(flat_w[order])
    dst_pair = jnp.zeros((T * K,), jnp.int32).at[order].set(dst_sorted)
    inv = dst_pair.reshape(T, K)
    block_start = jnp.arange(NB, dtype=jnp.int32) * BM
    block_expert = jnp.minimum(
        jnp.searchsorted(pad_end, block_start, side="right"), E - 1
    ).astype(jnp.int32)
    return row_token, row_w, inv, block_expert


def kernel(hidden_states, router_logits, w1, w3, w2):
    x = hidden_states.reshape(T, H)
    row_token, row_w, inv, block_expert = _plan(router_logits)

    sc_gather, sc_combine = _sc_kernels()
    x_pad = sc_gather(x, row_token.reshape(NW, G_NCH, G_CH))
    yw = _tc_mlp(block_expert, x_pad, w1, w3, w2, row_w.reshape(NB, 1, BM))
    out = sc_combine(
        yw,
        inv[:, 0].reshape(NW, C_NCH, C_CH),
        inv[:, 1].reshape(NW, C_NCH, C_CH),
    )
    return out


# speedup vs baseline: 1.2846x; 1.0338x over previous
import functools

import jax
import jax.numpy as jnp
from jax import lax
from jax.experimental import pallas as pl
from jax.experimental.pallas import tpu as pltpu
from jax.experimental.pallas import tpu_sc as plsc

E = 8
K = 2
H = 1024
I = 2048
T = 4096

BM = 256
NB = (T * K) // BM + E
P = NB * BM

NC = 2
NS = 16
NW = NC * NS

G_CH = 32
G_NCH = P // (NW * G_CH)
G_ROWS_W = P // NW
G_NBUF = 3


def _sc_gather_body(x_hbm, ids_hbm, out_hbm, idx_v, buf0, buf1, buf2,
                    g0, g1, g2, w0, w1s, w2s):
    wid = lax.axis_index("s") * NC + lax.axis_index("c")
    rbase = wid * G_ROWS_W
    pltpu.sync_copy(ids_hbm.at[wid], idx_v)
    bufs = (buf0, buf1, buf2)
    gsems = (g0, g1, g2)
    wsems = (w0, w1s, w2s)
    gcp = [None] * G_NBUF
    wcp = [None] * G_NBUF
    for c in range(min(2, G_NCH)):
        gcp[c] = pltpu.async_copy(x_hbm.at[idx_v.at[c]], bufs[c], gsems[c])
    for c in range(G_NCH):
        b = c % G_NBUF
        gcp[b].wait()
        wcp[b] = pltpu.async_copy(
            bufs[b], out_hbm.at[pl.ds(rbase + c * G_CH, G_CH)], wsems[b]
        )
        if c + 2 < G_NCH:
            nb = (c + 2) % G_NBUF
            if wcp[nb] is not None:
                wcp[nb].wait()
            gcp[nb] = pltpu.async_copy(
                x_hbm.at[idx_v.at[c + 2]], bufs[nb], gsems[nb]
            )
    for b in range(G_NBUF):
        if wcp[b] is not None:
            wcp[b].wait()


C_CH = 16
C_NCH = T // (NW * C_CH)
C_TOK_W = T // NW


C_NBUF = 3


def _sc_combine_body(yw_hbm, i0_hbm, i1_hbm, out_hbm, i0v, i1v,
                     a0, a1, a2, b0, b1, b2,
                     sa0, sa1, sa2, sb0, sb1, sb2, sw0, sw1, sw2):
    wid = lax.axis_index("s") * NC + lax.axis_index("c")
    tbase = wid * C_TOK_W
    pltpu.sync_copy(i0_hbm.at[wid], i0v)
    pltpu.sync_copy(i1_hbm.at[wid], i1v)
    abufs = (a0, a1, a2)
    bbufs = (b0, b1, b2)
    asems = (sa0, sa1, sa2)
    bsems = (sb0, sb1, sb2)
    wsems = (sw0, sw1, sw2)
    acp = [None] * C_NBUF
    bcp = [None] * C_NBUF
    wcp = [None] * C_NBUF
    for c in range(min(2, C_NCH)):
        acp[c] = pltpu.async_copy(yw_hbm.at[i0v.at[c]], abufs[c], asems[c])
        bcp[c] = pltpu.async_copy(yw_hbm.at[i1v.at[c]], bbufs[c], bsems[c])
    for c in range(C_NCH):
        b = c % C_NBUF
        acp[b].wait()
        bcp[b].wait()
        a_buf, b_buf = abufs[b], bbufs[b]
        for r in range(C_CH):
            def _add(j, _, r=r):
                sl = pl.ds(j * 16, 16)
                a_buf[r, sl] = a_buf[r, sl] + b_buf[r, sl]
                return 0
            lax.fori_loop(0, H // 16, _add, 0)
        wcp[b] = pltpu.async_copy(
            a_buf, out_hbm.at[pl.ds(tbase + c * C_CH, C_CH)], wsems[b]
        )
        if c + 2 < C_NCH:
            nb = (c + 2) % C_NBUF
            if wcp[nb] is not None:
                wcp[nb].wait()
            acp[nb] = pltpu.async_copy(yw_hbm.at[i0v.at[c + 2]], abufs[nb], asems[nb])
            bcp[nb] = pltpu.async_copy(yw_hbm.at[i1v.at[c + 2]], bbufs[nb], bsems[nb])
    for b in range(C_NBUF):
        if wcp[b] is not None:
            wcp[b].wait()


@functools.lru_cache(maxsize=None)
def _sc_kernels():
    mesh = plsc.VectorSubcoreMesh(
        core_axis_name="c", subcore_axis_name="s", num_cores=NC, num_subcores=NS
    )
    gather = pl.kernel(
        _sc_gather_body,
        out_type=jax.ShapeDtypeStruct((P, H), jnp.float32),
        mesh=mesh,
        scratch_types=[pltpu.VMEM((G_NCH, G_CH), jnp.int32)]
        + [pltpu.VMEM((G_CH, H), jnp.float32) for _ in range(G_NBUF)]
        + [pltpu.SemaphoreType.DMA] * (2 * G_NBUF),
    )
    combine = pl.kernel(
        _sc_combine_body,
        out_type=jax.ShapeDtypeStruct((T, H), jnp.float32),
        mesh=mesh,
        scratch_types=[pltpu.VMEM((C_NCH, C_CH), jnp.int32)] * 2
        + [pltpu.VMEM((C_CH, H), jnp.float32) for _ in range(2 * C_NBUF)]
        + [pltpu.SemaphoreType.DMA] * (3 * C_NBUF),
    )
    return gather, combine


def _mlp_body(s_ref, x_ref, w1_ref, w3_ref, w2_ref, rw_ref, out_ref):
    x = x_ref[...].astype(jnp.bfloat16)
    w1 = w1_ref[0].astype(jnp.bfloat16)
    w3 = w3_ref[0].astype(jnp.bfloat16)
    w2 = w2_ref[0].astype(jnp.bfloat16)
    dn = (((1,), (1,)), ((), ()))
    g = lax.dot_general(x, w1, dn, preferred_element_type=jnp.float32)
    u = lax.dot_general(x, w3, dn, preferred_element_type=jnp.float32)
    act = (g * jax.nn.sigmoid(g) * u).astype(jnp.bfloat16)
    y = lax.dot_general(act, w2, dn, preferred_element_type=jnp.float32)
    out_ref[...] = y * rw_ref[0, 0, :][:, None]


_tc_mlp = pl.pallas_call(
    _mlp_body,
    grid_spec=pltpu.PrefetchScalarGridSpec(
        num_scalar_prefetch=1,
        grid=(NB,),
        in_specs=[
            pl.BlockSpec((BM, H), lambda b, s: (b, 0)),
            pl.BlockSpec((1, I, H), lambda b, s: (s[b], 0, 0)),
            pl.BlockSpec((1, I, H), lambda b, s: (s[b], 0, 0)),
            pl.BlockSpec((1, H, I), lambda b, s: (s[b], 0, 0)),
            pl.BlockSpec((1, 1, BM), lambda b, s: (b, 0, 0)),
        ],
        out_specs=pl.BlockSpec((BM, H), lambda b, s: (b, 0)),
    ),
    out_shape=jax.ShapeDtypeStruct((P, H), jnp.float32),
    compiler_params=pltpu.CompilerParams(
        dimension_semantics=("arbitrary",),
        vmem_limit_bytes=100 * 1024 * 1024,
    ),
)


def _plan(router_logits):
    probs = jax.nn.softmax(router_logits, axis=-1)
    rw, sel = lax.top_k(probs, K)
    flat_e = sel.reshape(-1).astype(jnp.int32)
    flat_w = rw.reshape(-1).astype(jnp.float32)
    flat_t = (jnp.arange(T * K, dtype=jnp.int32) // K)
    counts = jnp.bincount(flat_e, length=E)
    pc = ((counts + BM - 1) // BM) * BM
    pad_end = jnp.cumsum(pc)
    pad_off = pad_end - pc
    off = jnp.cumsum(counts) - counts
    order = jnp.argsort(flat_e, stable=True)
    g_sorted = flat_e[order]
    s = jnp.arange(T * K, dtype=jnp.int32)
    dst_sorted = (pad_off[g_sorted] + s - off[g_sorted]).astype(jnp.int32)
    row_token = jnp.zeros((P,), jnp.int32).at[dst_sorted].set(flat_t[order])
    row_w = jnp.zeros((P,), jnp.float32).at[dst_sorted].set(flat_w[order])
    dst_pair = jnp.zeros((T * K,), jnp.int32).at[order].set(dst_sorted)
    inv = dst_pair.reshape(T, K)
    block_start = jnp.arange(NB, dtype=jnp.int32) * BM
    block_expert = jnp.minimum(
        jnp.searchsorted(pad_end, block_start, side="right"), E - 1
    ).astype(jnp.int32)
    return row_token, row_w, inv, block_expert


def kernel(hidden_states, router_logits, w1, w3, w2):
    x = hidden_states.reshape(T, H)
    row_token, row_w, inv, block_expert = _plan(router_logits)

    sc_gather, sc_combine = _sc_kernels()
    x_pad = sc_gather(x, row_token.reshape(NW, G_NCH, G_CH))
    yw = _tc_mlp(block_expert, x_pad, w1, w3, w2, row_w.reshape(NB, 1, BM))
    out = sc_combine(
        yw,
        inv[:, 0].reshape(NW, C_NCH, C_CH),
        inv[:, 1].reshape(NW, C_NCH, C_CH),
    )
    return out


# speedup vs baseline: 1.2975x; 1.0100x over previous
import functools

import jax
import jax.numpy as jnp
from jax import lax
from jax.experimental import pallas as pl
from jax.experimental.pallas import tpu as pltpu
from jax.experimental.pallas import tpu_sc as plsc

E = 8
K = 2
H = 1024
I = 2048
T = 4096

BM = 256
NB = (T * K) // BM + E
P = NB * BM

NC = 2
NS = 16
NW = NC * NS

G_CH = 32
G_NCH = P // (NW * G_CH)
G_ROWS_W = P // NW
G_NBUF = 3


def _sc_gather_body(x_hbm, ids_hbm, out_hbm, idx_v, buf0, buf1, buf2,
                    g0, g1, g2, w0, w1s, w2s):
    wid = lax.axis_index("s") * NC + lax.axis_index("c")
    rbase = wid * G_ROWS_W
    pltpu.sync_copy(ids_hbm.at[wid], idx_v)
    bufs = (buf0, buf1, buf2)
    gsems = (g0, g1, g2)
    wsems = (w0, w1s, w2s)
    gcp = [None] * G_NBUF
    wcp = [None] * G_NBUF
    for c in range(min(2, G_NCH)):
        gcp[c] = pltpu.async_copy(x_hbm.at[idx_v.at[c]], bufs[c], gsems[c])
    for c in range(G_NCH):
        b = c % G_NBUF
        gcp[b].wait()
        wcp[b] = pltpu.async_copy(
            bufs[b], out_hbm.at[pl.ds(rbase + c * G_CH, G_CH)], wsems[b]
        )
        if c + 2 < G_NCH:
            nb = (c + 2) % G_NBUF
            if wcp[nb] is not None:
                wcp[nb].wait()
            gcp[nb] = pltpu.async_copy(
                x_hbm.at[idx_v.at[c + 2]], bufs[nb], gsems[nb]
            )
    for b in range(G_NBUF):
        if wcp[b] is not None:
            wcp[b].wait()


C_CH = 16
C_NCH = T // (NW * C_CH)
C_TOK_W = T // NW


C_NBUF = 3


def _sc_combine_body(yw_hbm, i0_hbm, i1_hbm, out_hbm, i0v, i1v,
                     a0, a1, a2, b0, b1, b2,
                     sa0, sa1, sa2, sb0, sb1, sb2, sw0, sw1, sw2):
    wid = lax.axis_index("s") * NC + lax.axis_index("c")
    tbase = wid * C_TOK_W
    pltpu.sync_copy(i0_hbm.at[wid], i0v)
    pltpu.sync_copy(i1_hbm.at[wid], i1v)
    abufs = (a0, a1, a2)
    bbufs = (b0, b1, b2)
    asems = (sa0, sa1, sa2)
    bsems = (sb0, sb1, sb2)
    wsems = (sw0, sw1, sw2)
    acp = [None] * C_NBUF
    bcp = [None] * C_NBUF
    wcp = [None] * C_NBUF
    for c in range(min(2, C_NCH)):
        acp[c] = pltpu.async_copy(yw_hbm.at[i0v.at[c]], abufs[c], asems[c])
        bcp[c] = pltpu.async_copy(yw_hbm.at[i1v.at[c]], bbufs[c], bsems[c])
    for c in range(C_NCH):
        b = c % C_NBUF
        acp[b].wait()
        bcp[b].wait()
        a_buf, b_buf = abufs[b], bbufs[b]
        for r in range(C_CH):
            def _add(j, _, r=r):
                sl = pl.ds(j * 16, 16)
                a_buf[r, sl] = a_buf[r, sl] + b_buf[r, sl]
                return 0
            lax.fori_loop(0, H // 16, _add, 0)
        wcp[b] = pltpu.async_copy(
            a_buf, out_hbm.at[pl.ds(tbase + c * C_CH, C_CH)], wsems[b]
        )
        if c + 2 < C_NCH:
            nb = (c + 2) % C_NBUF
            if wcp[nb] is not None:
                wcp[nb].wait()
            acp[nb] = pltpu.async_copy(yw_hbm.at[i0v.at[c + 2]], abufs[nb], asems[nb])
            bcp[nb] = pltpu.async_copy(yw_hbm.at[i1v.at[c + 2]], bbufs[nb], bsems[nb])
    for b in range(C_NBUF):
        if wcp[b] is not None:
            wcp[b].wait()


@functools.lru_cache(maxsize=None)
def _sc_kernels():
    mesh = plsc.VectorSubcoreMesh(
        core_axis_name="c", subcore_axis_name="s", num_cores=NC, num_subcores=NS
    )
    gather = pl.kernel(
        _sc_gather_body,
        out_type=jax.ShapeDtypeStruct((P, 8, H // 8), jnp.float32),
        mesh=mesh,
        scratch_types=[pltpu.VMEM((G_NCH, G_CH), jnp.int32)]
        + [pltpu.VMEM((G_CH, 8, H // 8), jnp.float32) for _ in range(G_NBUF)]
        + [pltpu.SemaphoreType.DMA] * (2 * G_NBUF),
    )
    combine = pl.kernel(
        _sc_combine_body,
        out_type=jax.ShapeDtypeStruct((T, H), jnp.float32),
        mesh=mesh,
        scratch_types=[pltpu.VMEM((C_NCH, C_CH), jnp.int32)] * 2
        + [pltpu.VMEM((C_CH, H), jnp.float32) for _ in range(2 * C_NBUF)]
        + [pltpu.SemaphoreType.DMA] * (3 * C_NBUF),
    )
    return gather, combine


def _mlp_body(s_ref, x_ref, w1_ref, w3_ref, w2_ref, rw_ref, out_ref):
    x = x_ref[...].reshape(BM, H).astype(jnp.bfloat16)
    w1 = w1_ref[0].astype(jnp.bfloat16)
    w3 = w3_ref[0].astype(jnp.bfloat16)
    w2 = w2_ref[0].astype(jnp.bfloat16)
    dn = (((1,), (1,)), ((), ()))
    g = lax.dot_general(x, w1, dn, preferred_element_type=jnp.float32)
    u = lax.dot_general(x, w3, dn, preferred_element_type=jnp.float32)
    act = (g * jax.nn.sigmoid(g) * u).astype(jnp.bfloat16)
    y = lax.dot_general(act, w2, dn, preferred_element_type=jnp.float32)
    out_ref[...] = y * rw_ref[0, 0, :][:, None]


_tc_mlp = pl.pallas_call(
    _mlp_body,
    grid_spec=pltpu.PrefetchScalarGridSpec(
        num_scalar_prefetch=1,
        grid=(NB,),
        in_specs=[
            pl.BlockSpec((BM, 8, H // 8), lambda b, s: (b, 0, 0)),
            pl.BlockSpec((1, I, H), lambda b, s: (s[b], 0, 0)),
            pl.BlockSpec((1, I, H), lambda b, s: (s[b], 0, 0)),
            pl.BlockSpec((1, H, I), lambda b, s: (s[b], 0, 0)),
            pl.BlockSpec((1, 1, BM), lambda b, s: (b, 0, 0)),
        ],
        out_specs=pl.BlockSpec((BM, H), lambda b, s: (b, 0)),
    ),
    out_shape=jax.ShapeDtypeStruct((P, H), jnp.float32),
    compiler_params=pltpu.CompilerParams(
        dimension_semantics=("arbitrary",),
        vmem_limit_bytes=100 * 1024 * 1024,
    ),
)


def _plan(router_logits):
    probs = jax.nn.softmax(router_logits, axis=-1)
    rw, sel = lax.top_k(probs, K)
    flat_e = sel.reshape(-1).astype(jnp.int32)
    flat_w = rw.reshape(-1).astype(jnp.float32)
    flat_t = (jnp.arange(T * K, dtype=jnp.int32) // K)
    counts = jnp.bincount(flat_e, length=E)
    pc = ((counts + BM - 1) // BM) * BM
    pad_end = jnp.cumsum(pc)
    pad_off = pad_end - pc
    off = jnp.cumsum(counts) - counts
    order = jnp.argsort(flat_e, stable=True)
    g_sorted = flat_e[order]
    s = jnp.arange(T * K, dtype=jnp.int32)
    dst_sorted = (pad_off[g_sorted] + s - off[g_sorted]).astype(jnp.int32)
    row_token = jnp.zeros((P,), jnp.int32).at[dst_sorted].set(flat_t[order])
    row_w = jnp.zeros((P,), jnp.float32).at[dst_sorted].set(flat_w[order])
    dst_pair = jnp.zeros((T * K,), jnp.int32).at[order].set(dst_sorted)
    inv = dst_pair.reshape(T, K)
    block_start = jnp.arange(NB, dtype=jnp.int32) * BM
    block_expert = jnp.minimum(
        jnp.searchsorted(pad_end, block_start, side="right"), E - 1
    ).astype(jnp.int32)
    return row_token, row_w, inv, block_expert


def kernel(hidden_states, router_logits, w1, w3, w2):
    x = hidden_states.reshape(T, H)
    row_token, row_w, inv, block_expert = _plan(router_logits)

    sc_gather, sc_combine = _sc_kernels()
    x_pad = sc_gather(x.reshape(T, 8, H // 8), row_token.reshape(NW, G_NCH, G_CH))
    yw = _tc_mlp(block_expert, x_pad, w1, w3, w2, row_w.reshape(NB, 1, BM))
    out = sc_combine(
        yw,
        inv[:, 0].reshape(NW, C_NCH, C_CH),
        inv[:, 1].reshape(NW, C_NCH, C_CH),
    )
    return out
